# baseline (speedup 1.0000x reference)
import functools

import jax
import jax.numpy as jnp
from jax import lax
from jax.experimental import pallas as pl
from jax.experimental.pallas import tpu as pltpu
from jax.experimental.pallas import tpu_sc as plsc

_NC = 2
_NS = 16
_NW = _NC * _NS
_CH = 128
_L16 = 16
_NRB = 4
_AHEAD = 2

_params = pltpu.CompilerParams(use_tc_tiling_on_sc=True, needs_layout_passes=False)
_mesh = plsc.VectorSubcoreMesh(core_axis_name="c", subcore_axis_name="s")


def _diags():
    iota = jax.lax.iota(jnp.int32, _L16)
    return [jnp.bitwise_and(iota + k, _L16 - 1) for k in range(_L16)]


@functools.partial(jax.jit, static_argnames=("vocab", "emb"))
def _sc_repack(table_t, *, vocab, emb):
    lanes = 2 * emb
    ntc = (vocab + lanes - 1) // lanes
    vpad = lanes * ntc

    @functools.partial(
        pl.kernel,
        out_type=jax.ShapeDtypeStruct((vpad, lanes), jnp.float32),
        mesh=_mesh,
        scratch_types=[
            *[pltpu.VMEM((emb, lanes), jnp.float32) for _ in range(2)],
            *[pltpu.VMEM((lanes, lanes), jnp.float32) for _ in range(2)],
            *[pltpu.SemaphoreType.DMA for _ in range(4)],
        ],
        compiler_params=_params,
    )
    def body(tab_hbm, out_hbm, in0, in1, ob0, ob1, li0, li1, so0, so1):
        ins, obs = (in0, in1), (ob0, ob1)
        lsems, ssems = (li0, li1), (so0, so1)
        wid = lax.axis_index("s") * _NC + lax.axis_index("c")
        n_i = (ntc - wid + _NW - 1) // _NW
        diags = _diags()
        iota = jax.lax.iota(jnp.int32, _L16)

        def load_desc(i, b):
            tc = (wid + i * _NW) * lanes
            return pltpu.make_async_copy(
                tab_hbm.at[:, pl.ds(tc, lanes)], ins[b], lsems[b]
            )

        def store_desc(i, b):
            r0 = (wid + i * _NW) * lanes
            return pltpu.make_async_copy(
                obs[b], out_hbm.at[pl.ds(r0, lanes)], ssems[b]
            )

        @pl.when(n_i > 0)
        def _():
            load_desc(0, 0).start()

        @pl.loop(0, n_i)
        def _(i):
            for b in range(2):

                @pl.when((i & 1) == b)
                def _():
                    @pl.when(i + 1 < n_i)
                    def _():
                        load_desc(i + 1, 1 - b).start()

                    load_desc(i, b).wait()

                    @pl.when(i >= 2)
                    def _():
                        store_desc(0, b).wait()

                    @pl.loop(0, lanes // _L16)
                    def _(vg):
                        v_idx = iota + _L16 * vg
                        for cg in range(emb // _L16):
                            for k in range(_L16):
                                e_idx = diags[k] + _L16 * cg
                                vals = plsc.load_gather(ins[b], [e_idx, v_idx])
                                plsc.store_scatter(obs[b], [v_idx, e_idx], vals)

                    store_desc(i, b).start()

        @pl.when(n_i >= 1)
        def _():
            store_desc(0, 0).wait()

        @pl.when(n_i >= 2)
        def _():
            store_desc(0, 1).wait()

    return body(table_t)


@functools.partial(jax.jit, static_argnames=("seq_len", "emb"))
def _sc_embed(seq_t, table2, *, seq_len, emb):
    groups = _CH // _L16

    @functools.partial(
        pl.kernel,
        out_type=jax.ShapeDtypeStruct((seq_len, emb, _NW * _CH), jnp.float32),
        mesh=_mesh,
        scratch_types=[
            pltpu.VMEM((seq_len, _CH), jnp.int32),
            *[pltpu.VMEM((_CH, 2 * emb), jnp.float32) for _ in range(_NRB)],
            *[pltpu.VMEM((emb, _CH), jnp.float32) for _ in range(2)],
            *[pltpu.SemaphoreType.DMA for _ in range(_NRB + 2)],
        ],
        compiler_params=_params,
    )
    def body(seq_hbm, table_hbm, out_hbm, idx_v, *bufs_and_sems):
        rows = bufs_and_sems[:_NRB]
        slabs = bufs_and_sems[_NRB:_NRB + 2]
        gsems = bufs_and_sems[_NRB + 2:2 * _NRB + 2]
        ssems = bufs_and_sems[2 * _NRB + 2:]
        wid = lax.axis_index("s") * _NC + lax.axis_index("c")
        col0 = wid * _CH
        diags = _diags()
        iota = jax.lax.iota(jnp.int32, _L16)

        def gather_desc(l, rb):
            return pltpu.make_async_copy(
                table_hbm.at[idx_v.at[l]], rows[rb], gsems[rb]
            )

        def store_desc(l, sb):
            return pltpu.make_async_copy(
                slabs[sb], out_hbm.at[l, :, pl.ds(col0, _CH)], ssems[sb]
            )

        pltpu.sync_copy(seq_hbm.at[:, pl.ds(col0, _CH)], idx_v)
        for p in range(_AHEAD):
            gather_desc(p, p).start()

        @pl.loop(0, seq_len, step=_NRB)
        def _(l0):
            for k in range(_NRB):
                l = l0 + k
                sb = k % 2

                @pl.when(l + _AHEAD < seq_len)
                def _():
                    gather_desc(l + _AHEAD, (k + _AHEAD) % _NRB).start()

                gather_desc(l, k).wait()

                @pl.when(l >= 2)
                def _():
                    store_desc(0, sb).wait()

                @pl.loop(0, groups)
                def _(tg):
                    row_ids = iota + tg * _L16
                    for eg in range(emb // _L16):
                        for j in range(_L16):
                            e_idx = diags[j] + eg * _L16
                            vals = plsc.load_gather(rows[k], [row_ids, e_idx])
                            plsc.store_scatter(slabs[sb], [e_idx, row_ids], vals)

                store_desc(l, sb).start()

        store_desc(0, 0).wait()
        store_desc(0, 1).wait()

    return body(seq_t, table2)


def kernel(sequences, embedding_weight):
    b, l = sequences.shape
    v, emb = embedding_weight.shape
    seq_t = sequences.T.astype(jnp.int32)
    table_t = embedding_weight.T
    table2 = _sc_repack(table_t, vocab=v, emb=emb)
    out_t = _sc_embed(seq_t, table2, seq_len=l, emb=emb)
    return out_t.transpose(2, 0, 1)


# speedup vs baseline: 1.1016x; 1.1016x over previous
import functools

import jax
import jax.numpy as jnp
from jax import lax
from jax.experimental import pallas as pl
from jax.experimental.pallas import tpu as pltpu
from jax.experimental.pallas import tpu_sc as plsc

_NC = 2
_NS = 16
_NW = _NC * _NS
_CH = 128
_L16 = 16
_NRB = 4
_AHEAD = 2

_params = pltpu.CompilerParams(use_tc_tiling_on_sc=True, needs_layout_passes=False)
_mesh = plsc.VectorSubcoreMesh(core_axis_name="c", subcore_axis_name="s")


def _diags():
    iota = jax.lax.iota(jnp.int32, _L16)
    return [jnp.bitwise_and(iota + k, _L16 - 1) for k in range(_L16)]


@functools.partial(jax.jit, static_argnames=("vocab", "emb"))
def _sc_repack(table_t, *, vocab, emb):
    lanes = 2 * emb
    ntc = (vocab + lanes - 1) // lanes
    vpad = lanes * ntc

    @functools.partial(
        pl.kernel,
        out_type=jax.ShapeDtypeStruct((vpad // 2, lanes), jnp.float32),
        mesh=_mesh,
        scratch_types=[
            *[pltpu.VMEM((emb, lanes), jnp.float32) for _ in range(2)],
            *[pltpu.VMEM((emb, lanes), jnp.float32) for _ in range(2)],
            *[pltpu.SemaphoreType.DMA for _ in range(4)],
        ],
        compiler_params=_params,
    )
    def body(tab_hbm, out_hbm, in0, in1, ob0, ob1, li0, li1, so0, so1):
        ins, obs = (in0, in1), (ob0, ob1)
        lsems, ssems = (li0, li1), (so0, so1)
        wid = lax.axis_index("s") * _NC + lax.axis_index("c")
        n_i = (ntc - wid + _NW - 1) // _NW
        diags = _diags()
        iota = jax.lax.iota(jnp.int32, _L16)
        qpat = jnp.bitwise_and(iota, 7)
        hipat = jax.lax.shift_right_logical(iota, 3)
        vpat = 2 * qpat + hipat
        hi64 = hipat * emb

        def load_desc(i, b):
            tc = (wid + i * _NW) * lanes
            return pltpu.make_async_copy(
                tab_hbm.at[:, pl.ds(tc, lanes)], ins[b], lsems[b]
            )

        def store_desc(i, b):
            r0 = (wid + i * _NW) * emb
            return pltpu.make_async_copy(
                obs[b], out_hbm.at[pl.ds(r0, emb)], ssems[b]
            )

        @pl.when(n_i > 0)
        def _():
            load_desc(0, 0).start()

        @pl.loop(0, n_i)
        def _(i):
            for b in range(2):

                @pl.when((i & 1) == b)
                def _():
                    @pl.when(i + 1 < n_i)
                    def _():
                        load_desc(i + 1, 1 - b).start()

                    load_desc(i, b).wait()

                    @pl.when(i >= 2)
                    def _():
                        store_desc(0, b).wait()

                    @pl.loop(0, emb // 8)
                    def _(qg):
                        v_idx = vpat + 16 * qg
                        q_idx = qpat + 8 * qg
                        for eg in range(emb // _L16):
                            for k in range(_L16):
                                e_idx = diags[k] + _L16 * eg
                                vals = plsc.load_gather(ins[b], [e_idx, v_idx])
                                plsc.store_scatter(
                                    obs[b], [q_idx, hi64 + e_idx], vals
                                )

                    store_desc(i, b).start()

        @pl.when(n_i >= 1)
        def _():
            store_desc(0, 0).wait()

        @pl.when(n_i >= 2)
        def _():
            store_desc(0, 1).wait()

    return body(table_t)


@functools.partial(jax.jit, static_argnames=("seq_len", "emb"))
def _sc_embed(seq_t, table2, *, seq_len, emb):
    groups = _CH // _L16

    @functools.partial(
        pl.kernel,
        out_type=jax.ShapeDtypeStruct((seq_len, emb, _NW * _CH), jnp.float32),
        mesh=_mesh,
        scratch_types=[
            pltpu.VMEM((seq_len, _CH), jnp.int32),
            pltpu.VMEM((_NRB, _CH), jnp.int32),
            *[pltpu.VMEM((_CH, 2 * emb), jnp.float32) for _ in range(_NRB)],
            *[pltpu.VMEM((emb, _CH), jnp.float32) for _ in range(2)],
            *[pltpu.SemaphoreType.DMA for _ in range(_NRB + 2)],
        ],
        compiler_params=_params,
    )
    def body(seq_hbm, table_hbm, out_hbm, idx_v, half_v, *bufs_and_sems):
        rows = bufs_and_sems[:_NRB]
        slabs = bufs_and_sems[_NRB:_NRB + 2]
        gsems = bufs_and_sems[_NRB + 2:2 * _NRB + 2]
        ssems = bufs_and_sems[2 * _NRB + 2:]
        wid = lax.axis_index("s") * _NC + lax.axis_index("c")
        col0 = wid * _CH
        diags = _diags()
        iota = jax.lax.iota(jnp.int32, _L16)

        def gather_desc(l, rb, fire=False):
            if fire:
                for t in range(groups):
                    sl = pl.ds(t * _L16, _L16)
                    half_v[rb, sl] = jax.lax.shift_right_logical(
                        idx_v[l, sl], 1
                    )
            return pltpu.make_async_copy(
                table_hbm.at[half_v.at[rb]], rows[rb], gsems[rb]
            )

        def store_desc(l, sb):
            return pltpu.make_async_copy(
                slabs[sb], out_hbm.at[l, :, pl.ds(col0, _CH)], ssems[sb]
            )

        pltpu.sync_copy(seq_hbm.at[:, pl.ds(col0, _CH)], idx_v)
        for p in range(_AHEAD):
            gather_desc(p, p, fire=True).start()

        @pl.loop(0, seq_len, step=_NRB)
        def _(l0):
            for k in range(_NRB):
                l = l0 + k
                sb = k % 2

                @pl.when(l + _AHEAD < seq_len)
                def _():
                    gather_desc(
                        l + _AHEAD, (k + _AHEAD) % _NRB, fire=True
                    ).start()

                gather_desc(l, k).wait()

                @pl.when(l >= 2)
                def _():
                    store_desc(0, sb).wait()

                @pl.loop(0, groups)
                def _(tg):
                    tok = pl.ds(tg * _L16, _L16)
                    par = jax.lax.shift_left(
                        jnp.bitwise_and(idx_v[l, tok], 1), 6
                    )
                    row_ids = iota + tg * _L16
                    for eg in range(emb // _L16):
                        for j in range(_L16):
                            e_idx = diags[j] + eg * _L16
                            vals = plsc.load_gather(
                                rows[k], [row_ids, par + e_idx]
                            )
                            plsc.store_scatter(slabs[sb], [e_idx, row_ids], vals)

                store_desc(l, sb).start()

        store_desc(0, 0).wait()
        store_desc(0, 1).wait()

    return body(seq_t, table2)


def kernel(sequences, embedding_weight):
    b, l = sequences.shape
    v, emb = embedding_weight.shape
    seq_t = sequences.T.astype(jnp.int32)
    table_t = embedding_weight.T
    table2 = _sc_repack(table_t, vocab=v, emb=emb)
    out_t = _sc_embed(seq_t, table2, seq_len=l, emb=emb)
    return out_t.transpose(2, 0, 1)


# speedup vs baseline: 1.4253x; 1.2938x over previous
import functools

import jax
import jax.numpy as jnp
from jax import lax
from jax.experimental import pallas as pl
from jax.experimental.pallas import tpu as pltpu
from jax.experimental.pallas import tpu_sc as plsc

_NC = 2
_NS = 16
_NW = _NC * _NS
_CH = 128
_L16 = 16

_params = pltpu.CompilerParams(use_tc_tiling_on_sc=True, needs_layout_passes=False)
_mesh = plsc.VectorSubcoreMesh(core_axis_name="c", subcore_axis_name="s")


def _diags():
    iota = jax.lax.iota(jnp.int32, _L16)
    return [jnp.bitwise_and(iota + k, _L16 - 1) for k in range(_L16)]


@functools.partial(jax.jit, static_argnames=("vocab", "emb"))
def _sc_repack(table_t, *, vocab, emb):
    lanes = 2 * emb
    ntc = (vocab + lanes - 1) // lanes
    vpad = lanes * ntc

    @functools.partial(
        pl.kernel,
        out_type=jax.ShapeDtypeStruct((vpad // 2, lanes), jnp.float32),
        mesh=_mesh,
        scratch_types=[
            *[pltpu.VMEM((emb, lanes), jnp.float32) for _ in range(2)],
            *[pltpu.VMEM((emb, lanes), jnp.float32) for _ in range(2)],
            *[pltpu.SemaphoreType.DMA for _ in range(4)],
        ],
        compiler_params=_params,
    )
    def body(tab_hbm, out_hbm, in0, in1, ob0, ob1, li0, li1, so0, so1):
        ins, obs = (in0, in1), (ob0, ob1)
        lsems, ssems = (li0, li1), (so0, so1)
        wid = lax.axis_index("s") * _NC + lax.axis_index("c")
        n_i = (ntc - wid + _NW - 1) // _NW
        diags = _diags()
        iota = jax.lax.iota(jnp.int32, _L16)
        qpat = jnp.bitwise_and(iota, 7)
        hipat = jax.lax.shift_right_logical(iota, 3)
        vpat = 2 * qpat + hipat
        hi64 = hipat * emb

        def load_desc(i, b):
            tc = (wid + i * _NW) * lanes
            return pltpu.make_async_copy(
                tab_hbm.at[:, pl.ds(tc, lanes)], ins[b], lsems[b]
            )

        def store_desc(i, b):
            r0 = (wid + i * _NW) * emb
            return pltpu.make_async_copy(
                obs[b], out_hbm.at[pl.ds(r0, emb)], ssems[b]
            )

        @pl.when(n_i > 0)
        def _():
            load_desc(0, 0).start()

        @pl.loop(0, n_i)
        def _(i):
            for b in range(2):

                @pl.when((i & 1) == b)
                def _():
                    @pl.when(i + 1 < n_i)
                    def _():
                        load_desc(i + 1, 1 - b).start()

                    load_desc(i, b).wait()

                    @pl.when(i >= 2)
                    def _():
                        store_desc(0, b).wait()

                    @pl.loop(0, emb // 8)
                    def _(qg):
                        v_idx = vpat + _L16 * qg
                        q_idx = qpat + 8 * qg

                        @pl.loop(0, emb // _L16)
                        def _(eg):
                            for k in range(_L16):
                                e_idx = diags[k] + _L16 * eg
                                vals = plsc.load_gather(ins[b], [e_idx, v_idx])
                                plsc.store_scatter(
                                    obs[b], [q_idx, hi64 + e_idx], vals
                                )

                    store_desc(i, b).start()

        @pl.when(n_i >= 1)
        def _():
            store_desc(0, 0).wait()

        @pl.when(n_i >= 2)
        def _():
            store_desc(0, 1).wait()

    return body(table_t)


@functools.partial(jax.jit, static_argnames=("seq_len", "emb"))
def _sc_embed(seq_t, table2, *, seq_len, emb):
    groups = _CH // _L16

    @functools.partial(
        pl.kernel,
        out_type=jax.ShapeDtypeStruct((seq_len, emb, _NW * _CH), jnp.float32),
        mesh=_mesh,
        scratch_types=[
            pltpu.VMEM((seq_len, _CH), jnp.int32),
            pltpu.VMEM((2, _CH), jnp.int32),
            *[pltpu.VMEM((_CH, 2 * emb), jnp.float32) for _ in range(2)],
            *[pltpu.VMEM((emb, _CH), jnp.float32) for _ in range(2)],
            *[pltpu.SemaphoreType.DMA for _ in range(4)],
        ],
        compiler_params=_params,
    )
    def body(seq_hbm, table_hbm, out_hbm, idx_v, half_v, *bufs_and_sems):
        rows = bufs_and_sems[:2]
        slabs = bufs_and_sems[2:4]
        gsems = bufs_and_sems[4:6]
        ssems = bufs_and_sems[6:8]
        wid = lax.axis_index("s") * _NC + lax.axis_index("c")
        col0 = wid * _CH
        diags = _diags()
        iota = jax.lax.iota(jnp.int32, _L16)

        def fire_gather(l, rb):
            for t in range(groups):
                sl = pl.ds(t * _L16, _L16)
                half_v[rb, sl] = jax.lax.shift_right_logical(idx_v[l, sl], 1)
            pltpu.make_async_copy(
                table_hbm.at[half_v.at[rb]], rows[rb], gsems[rb]
            ).start()

        def wait_gather(rb):
            pltpu.make_async_copy(
                table_hbm.at[half_v.at[rb]], rows[rb], gsems[rb]
            ).wait()

        def store_desc(l, sb):
            return pltpu.make_async_copy(
                slabs[sb], out_hbm.at[l, :, pl.ds(col0, _CH)], ssems[sb]
            )

        pltpu.sync_copy(seq_hbm.at[:, pl.ds(col0, _CH)], idx_v)
        fire_gather(0, 0)

        @pl.loop(0, seq_len, step=2)
        def _(l0):
            for k in range(2):
                l = l0 + k

                @pl.when(l + 1 < seq_len)
                def _():
                    fire_gather(l + 1, 1 - k)

                wait_gather(k)

                @pl.when(l >= 2)
                def _():
                    store_desc(0, k).wait()

                @pl.loop(0, groups)
                def _(tg):
                    tok = pl.ds(tg * _L16, _L16)
                    par = jax.lax.shift_left(
                        jnp.bitwise_and(idx_v[l, tok], 1), 6
                    )
                    row_ids = iota + tg * _L16

                    @pl.loop(0, emb // _L16)
                    def _(eg):
                        for j in range(_L16):
                            e_idx = diags[j] + eg * _L16
                            vals = plsc.load_gather(
                                rows[k], [row_ids, par + e_idx]
                            )
                            plsc.store_scatter(
                                slabs[k], [e_idx, row_ids], vals
                            )

                store_desc(l, k).start()

        store_desc(0, 0).wait()
        store_desc(0, 1).wait()

    return body(seq_t, table2)


def kernel(sequences, embedding_weight):
    b, l = sequences.shape
    v, emb = embedding_weight.shape
    seq_t = sequences.T.astype(jnp.int32)
    table_t = embedding_weight.T
    table2 = _sc_repack(table_t, vocab=v, emb=emb)
    out_t = _sc_embed(seq_t, table2, seq_len=l, emb=emb)
    return out_t.transpose(2, 0, 1)
